# baseline (speedup 1.0000x reference)
import jax
import jax.numpy as jnp
from jax import lax
from jax.experimental import pallas as pl
from jax.experimental.pallas import tpu as pltpu
from jax.experimental.pallas import tpu_sc as plsc

NN = 10000
NP = 10240
EE = 320000
LL = 16
NC = 2
NS = 16
NW = NC * NS
ET = 10240
NB = ET // 128
GB = 8
NG = NB // GB
SL = NP // NS

_mesh = plsc.VectorSubcoreMesh(
    core_axis_name="c", subcore_axis_name="s", num_cores=NC, num_subcores=NS
)
_sc_params = pltpu.CompilerParams(
    needs_layout_passes=False, use_tc_tiling_on_sc=False
)


def _rsqrt_newton(x):
    i = plsc.bitcast(x, jnp.int32)
    i = jnp.int32(0x5F3759DF) - lax.shift_right_logical(i, 1)
    y = plsc.bitcast(i, jnp.float32)
    for _ in range(3):
        y = y * (1.5 - 0.5 * x * y * y)
    return y


def _agg_groups(hs, accs, srcv, dstv, normv, rows, semg, sems):
    def fire_gather(g, p):
        for j in range(GB):
            pltpu.async_copy(
                hs.at[srcv.at[g * GB + j]],
                rows.at[p, pl.ds(j * 128, 128)],
                semg,
            )

    def wait_gather(g, p):
        for j in range(GB):
            pltpu.make_async_copy(
                hs.at[srcv.at[g * GB + j]],
                rows.at[p, pl.ds(j * 128, 128)],
                semg,
            ).wait()

    def fire_scatter(g, p):
        for j in range(GB):
            pltpu.async_copy(
                rows.at[p, pl.ds(j * 128, 128)],
                accs.at[dstv.at[g * GB + j]],
                sems,
                add=True,
            )

    def wait_scatter(g, p):
        for j in range(GB):
            pltpu.make_async_copy(
                rows.at[p, pl.ds(j * 128, 128)],
                accs.at[dstv.at[g * GB + j]],
                sems,
            ).wait()

    fire_gather(0, 0)

    def gb(g, _):
        p = lax.rem(g, 3)
        pn = lax.rem(g + 1, 3)
        wait_gather(g, p)

        @pl.when(g >= 2)
        def _():
            wait_scatter(g - 2, pn)

        @pl.when(g < NG - 1)
        def _():
            fire_gather(g + 1, pn)

        def sb(jj, _):
            for kk in range(8):
                n16 = normv[g * GB + jj, pl.ds(kk * LL, LL)]
                for l in range(LL):
                    r = jj * 128 + kk * LL + l
                    rows[p, r, :] = rows[p, r, :] * n16[l]
            return 0

        lax.fori_loop(0, GB, sb, 0)
        fire_scatter(g, p)
        return 0

    lax.fori_loop(0, NG, gb, 0)
    wait_scatter(NG - 2, (NG - 2) % 3)
    wait_scatter(NG - 1, (NG - 1) % 3)


def _pagg_body(h_hbm, src_hbm, dst_hbm, ew_hbm, dis_hbm, p_hbm,
               srcv, dstv, normv, degl, idxb, dgt5, disl, rows, zb,
               degs, diss, accs, semp, semg, sems):
    cid = lax.axis_index("c")
    sid = lax.axis_index("s")
    w = cid * NS + sid
    zero = jnp.zeros((LL,), jnp.float32)

    def zdeg(j, _):
        for k in range(8):
            degl[j, pl.ds(k * LL, LL)] = zero
        return 0

    lax.fori_loop(0, NB, zdeg, 0)
    iota = lax.iota(jnp.int32, LL)
    for k in range(NB // LL):
        idxb[pl.ds(k * LL, LL)] = iota + (k * LL)

    @pl.when(sid == 0)
    def _():
        pltpu.sync_copy(degl, degs)

    plsc.subcore_barrier()

    for q in range(2):
        c = 2 * sid + q
        pltpu.async_copy(dst_hbm.at[c], srcv, semp)
        pltpu.async_copy(ew_hbm.at[c], normv, semp)
        pltpu.make_async_copy(dst_hbm.at[c], srcv, semp).wait()
        pltpu.make_async_copy(ew_hbm.at[c], normv, semp).wait()

        def eb(j, _):
            for k in range(8):
                d16 = srcv[j, pl.ds(k * LL, LL)]
                w16 = normv[j, pl.ds(k * LL, LL)]
                plsc.addupdate_scatter(
                    degl,
                    [lax.shift_right_logical(d16, 7), jnp.bitwise_and(d16, 127)],
                    w16,
                )
            return 0

        lax.fori_loop(0, NB, eb, 0)

    pltpu.sync_copy(degl, degs.at[idxb], add=True)
    plsc.subcore_barrier()
    pltpu.sync_copy(degs.at[pl.ds(sid * (SL // 128), SL // 128)], dgt5)

    for i in range(SL // LL):
        acc = dgt5[i // 8, pl.ds((i % 8) * LL, LL)] + 1.0
        d = _rsqrt_newton(acc)
        disl[pl.ds(sid * SL + i * LL, LL)] = d
    pltpu.sync_copy(disl.at[pl.ds(sid * SL, SL)], diss.at[pl.ds(sid * SL, SL)])

    @pl.when(cid == 0)
    def _():
        pltpu.sync_copy(
            disl.at[pl.ds(sid * SL, SL)], dis_hbm.at[pl.ds(sid * SL, SL)]
        )

    plsc.subcore_barrier()
    pltpu.sync_copy(diss, disl)

    pltpu.async_copy(src_hbm.at[w], srcv, semp)
    pltpu.async_copy(dst_hbm.at[w], dstv, semp)
    pltpu.async_copy(ew_hbm.at[w], normv, semp)
    pltpu.async_copy(h_hbm.at[pl.ds(sid * SL, SL)], rows.at[0, pl.ds(0, SL)], semp)
    pltpu.make_async_copy(src_hbm.at[w], srcv, semp).wait()
    pltpu.make_async_copy(dst_hbm.at[w], dstv, semp).wait()
    pltpu.make_async_copy(ew_hbm.at[w], normv, semp).wait()
    pltpu.make_async_copy(h_hbm.at[pl.ds(sid * SL, SL)], rows.at[0, pl.ds(0, SL)], semp).wait()

    def nb(j, _):
        for k in range(8):
            s16 = srcv[j, pl.ds(k * LL, LL)]
            d16 = dstv[j, pl.ds(k * LL, LL)]
            w16 = normv[j, pl.ds(k * LL, LL)]
            a = plsc.load_gather(disl, [s16])
            b = plsc.load_gather(disl, [d16])
            normv[j, pl.ds(k * LL, LL)] = a * w16 * b
        return 0

    lax.fori_loop(0, NB, nb, 0)

    def z0(i, _):
        zb[i, :] = zero
        return 0

    lax.fori_loop(0, SL, z0, 0)

    @pl.when(cid == 0)
    def _():
        def s0(ii, _):
            d16 = disl[pl.ds(sid * SL + ii * LL, LL)]
            i16 = d16 * d16
            for l in range(LL):
                r = ii * LL + l
                zb[r, :] = rows[0, r, :] * i16[l]
            return 0

        lax.fori_loop(0, SL // LL, s0, 0)

    pltpu.sync_copy(zb, accs.at[pl.ds(sid * SL, SL)])
    plsc.subcore_barrier()

    _agg_groups(h_hbm, accs, srcv, dstv, normv, rows, semg, sems)
    plsc.subcore_barrier()
    pltpu.sync_copy(accs.at[pl.ds(sid * SL, SL)], p_hbm.at[cid, pl.ds(sid * SL, SL)])


_pagg_call = pl.kernel(
    _pagg_body,
    out_type=(
        jax.ShapeDtypeStruct((NP,), jnp.float32),
        jax.ShapeDtypeStruct((NC, NP, LL), jnp.float32),
    ),
    mesh=_mesh,
    compiler_params=_sc_params,
    scratch_types=[
        pltpu.VMEM((NB, 128), jnp.int32),
        pltpu.VMEM((NB, 128), jnp.int32),
        pltpu.VMEM((NB, 128), jnp.float32),
        pltpu.VMEM((NB, 128), jnp.float32),
        pltpu.VMEM((NB,), jnp.int32),
        pltpu.VMEM((SL // 128, 128), jnp.float32),
        pltpu.VMEM((NP,), jnp.float32),
        pltpu.VMEM((3, GB * 128, LL), jnp.float32),
        pltpu.VMEM((SL, LL), jnp.float32),
        pltpu.VMEM_SHARED((NB, 128), jnp.float32),
        pltpu.VMEM_SHARED((NP,), jnp.float32),
        pltpu.VMEM_SHARED((NP, LL), jnp.float32),
        pltpu.SemaphoreType.DMA,
        pltpu.SemaphoreType.DMA,
        pltpu.SemaphoreType.DMA,
    ],
)


def _agg_body(h_hbm, src_hbm, dst_hbm, ew_hbm, dis_hbm, p_hbm,
              srcv, dstv, normv, disl, rows, zb, hs, accs, semg, sems, seml):
    cid = lax.axis_index("c")
    sid = lax.axis_index("s")
    w = cid * NS + sid
    zero = jnp.zeros((LL,), jnp.float32)

    pltpu.async_copy(
        h_hbm.at[pl.ds(sid * SL, SL)], hs.at[pl.ds(sid * SL, SL)], seml
    )
    pltpu.async_copy(h_hbm.at[pl.ds(sid * SL, SL)], rows.at[0, pl.ds(0, SL)], seml)
    pltpu.async_copy(dis_hbm, disl, seml)
    pltpu.async_copy(src_hbm.at[w], srcv, seml)
    pltpu.async_copy(dst_hbm.at[w], dstv, seml)
    pltpu.async_copy(ew_hbm.at[w], normv, seml)

    def z0(i, _):
        zb[i, :] = zero
        return 0

    lax.fori_loop(0, SL, z0, 0)
    pltpu.make_async_copy(
        h_hbm.at[pl.ds(sid * SL, SL)], hs.at[pl.ds(sid * SL, SL)], seml
    ).wait()
    pltpu.make_async_copy(h_hbm.at[pl.ds(sid * SL, SL)], rows.at[0, pl.ds(0, SL)], seml).wait()
    pltpu.make_async_copy(dis_hbm, disl, seml).wait()
    pltpu.make_async_copy(src_hbm.at[w], srcv, seml).wait()
    pltpu.make_async_copy(dst_hbm.at[w], dstv, seml).wait()
    pltpu.make_async_copy(ew_hbm.at[w], normv, seml).wait()

    def nb(j, _):
        for k in range(8):
            s16 = srcv[j, pl.ds(k * LL, LL)]
            d16 = dstv[j, pl.ds(k * LL, LL)]
            w16 = normv[j, pl.ds(k * LL, LL)]
            a = plsc.load_gather(disl, [s16])
            b = plsc.load_gather(disl, [d16])
            normv[j, pl.ds(k * LL, LL)] = a * w16 * b
        return 0

    lax.fori_loop(0, NB, nb, 0)

    @pl.when(cid == 0)
    def _():
        def s0(ii, _):
            d16 = disl[pl.ds(sid * SL + ii * LL, LL)]
            i16 = d16 * d16
            for l in range(LL):
                r = ii * LL + l
                zb[r, :] = rows[0, r, :] * i16[l]
            return 0

        lax.fori_loop(0, SL // LL, s0, 0)

    pltpu.sync_copy(zb, accs.at[pl.ds(sid * SL, SL)])
    plsc.subcore_barrier()

    _agg_groups(hs, accs, srcv, dstv, normv, rows, semg, sems)
    plsc.subcore_barrier()
    pltpu.sync_copy(accs.at[pl.ds(sid * SL, SL)], p_hbm.at[cid, pl.ds(sid * SL, SL)])


_agg_call = pl.kernel(
    _agg_body,
    out_type=jax.ShapeDtypeStruct((NC, NP, LL), jnp.float32),
    mesh=_mesh,
    compiler_params=_sc_params,
    scratch_types=[
        pltpu.VMEM((NB, 128), jnp.int32),
        pltpu.VMEM((NB, 128), jnp.int32),
        pltpu.VMEM((NB, 128), jnp.float32),
        pltpu.VMEM((NP,), jnp.float32),
        pltpu.VMEM((3, GB * 128, LL), jnp.float32),
        pltpu.VMEM((SL, LL), jnp.float32),
        pltpu.VMEM_SHARED((NP, LL), jnp.float32),
        pltpu.VMEM_SHARED((NP, LL), jnp.float32),
        pltpu.SemaphoreType.DMA,
        pltpu.SemaphoreType.DMA,
        pltpu.SemaphoreType.DMA,
    ],
)


def _mm1_body(x_ref, w_ref, o_ref):
    o_ref[:] = jnp.dot(x_ref[:], w_ref[:], preferred_element_type=jnp.float32)


_mm1_call = pl.pallas_call(
    _mm1_body, out_shape=jax.ShapeDtypeStruct((NP, LL), jnp.float32)
)


def _comb_body(p_ref, b_ref, w_ref, o_ref):
    z = jnp.maximum(p_ref[0] + p_ref[1] + b_ref[:], 0.0)
    o_ref[:] = jnp.dot(z, w_ref[:], preferred_element_type=jnp.float32)


_comb_call = pl.pallas_call(
    _comb_body, out_shape=jax.ShapeDtypeStruct((NP, LL), jnp.float32)
)


def _final_body(p_ref, b_ref, w_ref, bl_ref, o_ref):
    z = jnp.maximum(p_ref[0] + p_ref[1] + b_ref[:], 0.0)
    y = jnp.dot(z, w_ref[:], preferred_element_type=jnp.float32) + bl_ref[:]
    o_ref[:] = 1.0 / (1.0 + jnp.exp(-y))


_final_call = pl.pallas_call(
    _final_body, out_shape=jax.ShapeDtypeStruct((NP, 8), jnp.float32)
)


def _pad_chunks(a, dtype):
    a = a.astype(dtype).reshape(NW, EE // NW)
    a = jnp.pad(a, ((0, 0), (0, ET - EE // NW)))
    return a.reshape(NW, NB, 128)


@jax.jit
def kernel(X, edge_index, edge_weight, W1, b1, W2, b2, W3, b3, Wl, bl):
    srcp = _pad_chunks(edge_index[0], jnp.int32)
    dstp = _pad_chunks(edge_index[1], jnp.int32)
    ewp = _pad_chunks(edge_weight, jnp.float32)
    Xp = jnp.pad(X, ((0, NP - NN), (0, 0)))
    W1p = jnp.pad(W1, ((0, 0), (0, LL - 12)))
    W2p = jnp.pad(W2, ((0, LL - 12), (0, LL - 6)))
    W3p = jnp.pad(W3, ((0, LL - 6), (0, LL - 3)))
    Wlp = jnp.pad(Wl, ((0, LL - 3), (0, 8 - 1)))
    b1p = jnp.pad(b1, (0, LL - 12)).reshape(1, LL)
    b2p = jnp.pad(b2, (0, LL - 6)).reshape(1, LL)
    b3p = jnp.pad(b3, (0, LL - 3)).reshape(1, LL)
    blp = jnp.pad(bl, (0, 8 - 1)).reshape(1, 8)

    h1 = _mm1_call(Xp, W1p)
    dis, p1 = _pagg_call(h1, srcp, dstp, ewp)
    h2 = _comb_call(p1, b1p, W2p)
    p2 = _agg_call(h2, srcp, dstp, ewp, dis)
    h3 = _comb_call(p2, b2p, W3p)
    p3 = _agg_call(h3, srcp, dstp, ewp, dis)
    out = _final_call(p3, b3p, Wlp, blp)
    return out[:NN, 0]


# speedup vs baseline: 1.1352x; 1.1352x over previous
import jax
import jax.numpy as jnp
from jax import lax
from jax.experimental import pallas as pl
from jax.experimental.pallas import tpu as pltpu
from jax.experimental.pallas import tpu_sc as plsc

NN = 10000
NP = 10240
EE = 320000
LL = 16
NC = 2
NS = 16
NW = NC * NS
ET = 10240
NB = ET // 128
GB = 8
NG = NB // GB
SL = NP // NS

_mesh = plsc.VectorSubcoreMesh(
    core_axis_name="c", subcore_axis_name="s", num_cores=NC, num_subcores=NS
)
_sc_params = pltpu.CompilerParams(
    needs_layout_passes=False, use_tc_tiling_on_sc=False
)


def _rsqrt_newton(x):
    i = plsc.bitcast(x, jnp.int32)
    i = jnp.int32(0x5F3759DF) - lax.shift_right_logical(i, 1)
    y = plsc.bitcast(i, jnp.float32)
    for _ in range(3):
        y = y * (1.5 - 0.5 * x * y * y)
    return y


def _agg_groups(hs, accs, srcv, dstv, normv, rows, semg, sems):
    def fire_gather(g, p):
        for j in range(GB):
            pltpu.async_copy(
                hs.at[srcv.at[g * GB + j]],
                rows.at[p, pl.ds(j * 128, 128)],
                semg,
            )

    def wait_gather(g, p):
        for j in range(GB):
            pltpu.make_async_copy(
                hs.at[srcv.at[g * GB + j]],
                rows.at[p, pl.ds(j * 128, 128)],
                semg,
            ).wait()

    def fire_scatter(g, p):
        for j in range(GB):
            pltpu.async_copy(
                rows.at[p, pl.ds(j * 128, 128)],
                accs.at[dstv.at[g * GB + j]],
                sems,
                add=True,
            )

    def wait_scatter(g, p):
        for j in range(GB):
            pltpu.make_async_copy(
                rows.at[p, pl.ds(j * 128, 128)],
                accs.at[dstv.at[g * GB + j]],
                sems,
            ).wait()

    fire_gather(0, 0)

    def gb(g, _):
        p = lax.rem(g, 3)
        pn = lax.rem(g + 1, 3)
        wait_gather(g, p)

        @pl.when(g >= 2)
        def _():
            wait_scatter(g - 2, pn)

        @pl.when(g < NG - 1)
        def _():
            fire_gather(g + 1, pn)

        def sb(jj, _):
            for kk in range(8):
                n16 = normv[g * GB + jj, pl.ds(kk * LL, LL)]
                for l in range(LL):
                    r = jj * 128 + kk * LL + l
                    rows[p, r, :] = rows[p, r, :] * n16[l]
            return 0

        lax.fori_loop(0, GB, sb, 0)
        fire_scatter(g, p)
        return 0

    lax.fori_loop(0, NG, gb, 0)
    wait_scatter(NG - 2, (NG - 2) % 3)
    wait_scatter(NG - 1, (NG - 1) % 3)


def _prep_body(src_hbm, dst_hbm, ew_hbm, dis_hbm,
               srcv, normv, degl, idxb, dgt5, disl, degs, semp):
    cid = lax.axis_index("c")
    sid = lax.axis_index("s")
    zero = jnp.zeros((LL,), jnp.float32)

    def zdeg(j, _):
        for k in range(8):
            degl[j, pl.ds(k * LL, LL)] = zero
        return 0

    lax.fori_loop(0, NB, zdeg, 0)
    iota = lax.iota(jnp.int32, LL)
    for k in range(NB // LL):
        idxb[pl.ds(k * LL, LL)] = iota + (k * LL)

    @pl.when(sid == 0)
    def _():
        pltpu.sync_copy(degl, degs)

    plsc.subcore_barrier()

    for q in range(2):
        c = 2 * sid + q
        pltpu.async_copy(dst_hbm.at[c], srcv, semp)
        pltpu.async_copy(ew_hbm.at[c], normv, semp)
        pltpu.make_async_copy(dst_hbm.at[c], srcv, semp).wait()
        pltpu.make_async_copy(ew_hbm.at[c], normv, semp).wait()

        def eb(j, _):
            for k in range(8):
                d16 = srcv[j, pl.ds(k * LL, LL)]
                w16 = normv[j, pl.ds(k * LL, LL)]
                plsc.addupdate_scatter(
                    degl,
                    [lax.shift_right_logical(d16, 7), jnp.bitwise_and(d16, 127)],
                    w16,
                )
            return 0

        lax.fori_loop(0, NB, eb, 0)

    pltpu.sync_copy(degl, degs.at[idxb], add=True)
    plsc.subcore_barrier()

    @pl.when(cid == 0)
    def _():
        pltpu.sync_copy(degs.at[pl.ds(sid * (SL // 128), SL // 128)], dgt5)
        for i in range(SL // LL):
            acc = dgt5[i // 8, pl.ds((i % 8) * LL, LL)] + 1.0
            d = _rsqrt_newton(acc)
            disl[pl.ds(i * LL, LL)] = d
        pltpu.sync_copy(disl, dis_hbm.at[pl.ds(sid * SL, SL)])


_prep_call = pl.kernel(
    _prep_body,
    out_type=jax.ShapeDtypeStruct((NP,), jnp.float32),
    mesh=_mesh,
    compiler_params=_sc_params,
    scratch_types=[
        pltpu.VMEM((NB, 128), jnp.int32),
        pltpu.VMEM((NB, 128), jnp.float32),
        pltpu.VMEM((NB, 128), jnp.float32),
        pltpu.VMEM((NB,), jnp.int32),
        pltpu.VMEM((SL // 128, 128), jnp.float32),
        pltpu.VMEM((SL,), jnp.float32),
        pltpu.VMEM_SHARED((NB, 128), jnp.float32),
        pltpu.SemaphoreType.DMA,
    ],
)


def _agg_body(h_hbm, src_hbm, dst_hbm, ew_hbm, dis_hbm, p_hbm,
              srcv, dstv, normv, disl, rows, zb, hs, accs, semg, sems, seml):
    cid = lax.axis_index("c")
    sid = lax.axis_index("s")
    w = cid * NS + sid
    zero = jnp.zeros((LL,), jnp.float32)

    pltpu.async_copy(
        h_hbm.at[pl.ds(sid * SL, SL)], hs.at[pl.ds(sid * SL, SL)], seml
    )
    pltpu.async_copy(h_hbm.at[pl.ds(sid * SL, SL)], rows.at[0, pl.ds(0, SL)], seml)
    pltpu.async_copy(dis_hbm, disl, seml)
    pltpu.async_copy(src_hbm.at[w], srcv, seml)
    pltpu.async_copy(dst_hbm.at[w], dstv, seml)
    pltpu.async_copy(ew_hbm.at[w], normv, seml)

    def z0(i, _):
        zb[i, :] = zero
        return 0

    lax.fori_loop(0, SL, z0, 0)
    pltpu.make_async_copy(
        h_hbm.at[pl.ds(sid * SL, SL)], hs.at[pl.ds(sid * SL, SL)], seml
    ).wait()
    pltpu.make_async_copy(h_hbm.at[pl.ds(sid * SL, SL)], rows.at[0, pl.ds(0, SL)], seml).wait()
    pltpu.make_async_copy(dis_hbm, disl, seml).wait()
    pltpu.make_async_copy(src_hbm.at[w], srcv, seml).wait()
    pltpu.make_async_copy(dst_hbm.at[w], dstv, seml).wait()
    pltpu.make_async_copy(ew_hbm.at[w], normv, seml).wait()

    def nb(j, _):
        for k in range(8):
            s16 = srcv[j, pl.ds(k * LL, LL)]
            d16 = dstv[j, pl.ds(k * LL, LL)]
            w16 = normv[j, pl.ds(k * LL, LL)]
            a = plsc.load_gather(disl, [s16])
            b = plsc.load_gather(disl, [d16])
            normv[j, pl.ds(k * LL, LL)] = a * w16 * b
        return 0

    lax.fori_loop(0, NB, nb, 0)

    @pl.when(cid == 0)
    def _():
        def s0(ii, _):
            d16 = disl[pl.ds(sid * SL + ii * LL, LL)]
            i16 = d16 * d16
            for l in range(LL):
                r = ii * LL + l
                zb[r, :] = rows[0, r, :] * i16[l]
            return 0

        lax.fori_loop(0, SL // LL, s0, 0)

    pltpu.sync_copy(zb, accs.at[pl.ds(sid * SL, SL)])
    plsc.subcore_barrier()

    _agg_groups(hs, accs, srcv, dstv, normv, rows, semg, sems)
    plsc.subcore_barrier()
    pltpu.sync_copy(accs.at[pl.ds(sid * SL, SL)], p_hbm.at[cid, pl.ds(sid * SL, SL)])


_agg_call = pl.kernel(
    _agg_body,
    out_type=jax.ShapeDtypeStruct((NC, NP, LL), jnp.float32),
    mesh=_mesh,
    compiler_params=_sc_params,
    scratch_types=[
        pltpu.VMEM((NB, 128), jnp.int32),
        pltpu.VMEM((NB, 128), jnp.int32),
        pltpu.VMEM((NB, 128), jnp.float32),
        pltpu.VMEM((NP,), jnp.float32),
        pltpu.VMEM((3, GB * 128, LL), jnp.float32),
        pltpu.VMEM((SL, LL), jnp.float32),
        pltpu.VMEM_SHARED((NP, LL), jnp.float32),
        pltpu.VMEM_SHARED((NP, LL), jnp.float32),
        pltpu.SemaphoreType.DMA,
        pltpu.SemaphoreType.DMA,
        pltpu.SemaphoreType.DMA,
    ],
)


def _mm1_body(x_ref, w_ref, o_ref):
    o_ref[:] = jnp.dot(x_ref[:], w_ref[:], preferred_element_type=jnp.float32)


_mm1_call = pl.pallas_call(
    _mm1_body, out_shape=jax.ShapeDtypeStruct((NP, LL), jnp.float32)
)


def _comb_body(p_ref, b_ref, w_ref, o_ref):
    z = jnp.maximum(p_ref[0] + p_ref[1] + b_ref[:], 0.0)
    o_ref[:] = jnp.dot(z, w_ref[:], preferred_element_type=jnp.float32)


_comb_call = pl.pallas_call(
    _comb_body, out_shape=jax.ShapeDtypeStruct((NP, LL), jnp.float32)
)


def _final_body(p_ref, b_ref, w_ref, bl_ref, o_ref):
    z = jnp.maximum(p_ref[0] + p_ref[1] + b_ref[:], 0.0)
    y = jnp.dot(z, w_ref[:], preferred_element_type=jnp.float32) + bl_ref[:]
    o_ref[:] = 1.0 / (1.0 + jnp.exp(-y))


_final_call = pl.pallas_call(
    _final_body, out_shape=jax.ShapeDtypeStruct((NP, 8), jnp.float32)
)


def _pad_chunks(a, dtype):
    a = a.astype(dtype).reshape(NW, EE // NW)
    a = jnp.pad(a, ((0, 0), (0, ET - EE // NW)))
    return a.reshape(NW, NB, 128)


@jax.jit
def kernel(X, edge_index, edge_weight, W1, b1, W2, b2, W3, b3, Wl, bl):
    srcp = _pad_chunks(edge_index[0], jnp.int32)
    dstp = _pad_chunks(edge_index[1], jnp.int32)
    ewp = _pad_chunks(edge_weight, jnp.float32)
    Xp = jnp.pad(X, ((0, NP - NN), (0, 0)))
    W1p = jnp.pad(W1, ((0, 0), (0, LL - 12)))
    W2p = jnp.pad(W2, ((0, LL - 12), (0, LL - 6)))
    W3p = jnp.pad(W3, ((0, LL - 6), (0, LL - 3)))
    Wlp = jnp.pad(Wl, ((0, LL - 3), (0, 8 - 1)))
    b1p = jnp.pad(b1, (0, LL - 12)).reshape(1, LL)
    b2p = jnp.pad(b2, (0, LL - 6)).reshape(1, LL)
    b3p = jnp.pad(b3, (0, LL - 3)).reshape(1, LL)
    blp = jnp.pad(bl, (0, 8 - 1)).reshape(1, 8)

    dis = _prep_call(srcp, dstp, ewp)
    h1 = _mm1_call(Xp, W1p)
    p1 = _agg_call(h1, srcp, dstp, ewp, dis)
    h2 = _comb_call(p1, b1p, W2p)
    p2 = _agg_call(h2, srcp, dstp, ewp, dis)
    h3 = _comb_call(p2, b2p, W3p)
    p3 = _agg_call(h3, srcp, dstp, ewp, dis)
    out = _final_call(p3, b3p, Wlp, blp)
    return out[:NN, 0]
